# baseline (speedup 1.0000x reference)
import functools

import jax
import jax.numpy as jnp
from jax import lax
from jax.experimental import pallas as pl
from jax.experimental.pallas import tpu as pltpu
from jax.experimental.pallas import tpu_sc as plsc

N = 10000
E = 160000
R = 65
F = 128
H = 128
G = 512
MLP_H = 64

ETOT = E + N
EIN = 172032
KR = 512
NTR = EIN // KR
T = 256
CHUNK = 128
EPAD = 196608
EPADX = EPAD + 4096
NT = EPAD // T
NCHUNK = EPAD // CHUNK
NSUB = 16
NW = 2 * NSUB
CPW = NCHUNK // NW
GRP = 3
NG = CPW // GRP
GRS = 2
CPS2 = NCHUNK // NSUB
NGS2 = CPS2 // GRS
SUPB = 4096
NSUP = EPAD // SUPB
SUBT = SUPB // T
NBCH = EIN // CHUNK
BPS = NBCH // NSUB
BPS2 = BPS // 2
HS = 325632
HST = 327680
MSEG = 2 * HS
HZR = HST // NSUB
HCP = HS // NSUB
NPAD = 10240
NHALF = NPAD // 2
ACCR = NHALF + 128
AZR = ACCR // NSUB
ACR = NHALF // NSUB
_UROWS = 1000


@functools.lru_cache(maxsize=None)
def _mesh():
    return plsc.VectorSubcoreMesh(core_axis_name="c", subcore_axis_name="s")


def _rank_body(et_ref, rank_ref, cnt_ref, base_ref):
    t = pl.program_id(0)

    @pl.when(t == 0)
    def _():
        base_ref[...] = jnp.zeros_like(base_ref)

    oh = (et_ref[...] == lax.broadcasted_iota(
        jnp.int32, (1, 128), 1)).astype(jnp.float32)
    ii = lax.broadcasted_iota(jnp.int32, (KR, KR), 0)
    jj = lax.broadcasted_iota(jnp.int32, (KR, KR), 1)
    ls = (ii > jj).astype(jnp.float32)
    pref = jnp.dot(ls, oh, preferred_element_type=jnp.float32)
    prefb = pref + base_ref[...]
    rank_ref[...] = jnp.sum(prefb * oh, axis=1,
                            keepdims=True).astype(jnp.int32)
    base_ref[...] = base_ref[...] + jnp.sum(oh, axis=0, keepdims=True)

    @pl.when(t == NTR - 1)
    def _():
        cnt_ref[...] = base_ref[...] + jnp.zeros_like(base_ref)


def _rank(et2):
    return pl.pallas_call(
        _rank_body,
        grid=(NTR,),
        in_specs=[pl.BlockSpec((KR, 1), lambda t: (t, 0))],
        out_specs=[pl.BlockSpec((KR, 1), lambda t: (t, 0)),
                   pl.BlockSpec((1, 128), lambda t: (0, 0))],
        out_shape=[jax.ShapeDtypeStruct((EIN, 1), jnp.int32),
                   jax.ShapeDtypeStruct((1, 128), jnp.float32)],
        scratch_shapes=[pltpu.VMEM((1, 128), jnp.float32)],
    )(et2)


@functools.lru_cache(maxsize=None)
def _bin_kernel():
  return functools.partial(
    pl.kernel,
    out_type=[jax.ShapeDtypeStruct((EPADX,), jnp.int32),
              jax.ShapeDtypeStruct((EPADX,), jnp.int32),
              jax.ShapeDtypeStruct((EPADX,), jnp.int32),
              jax.ShapeDtypeStruct((MSEG,), jnp.float32)],
    mesh=_mesh(),
    scratch_types=[
        pltpu.VMEM((CHUNK,), jnp.int32),
        pltpu.VMEM((CHUNK,), jnp.int32),
        pltpu.VMEM((CHUNK,), jnp.int32),
        pltpu.VMEM((CHUNK,), jnp.int32),
        pltpu.VMEM((CHUNK,), jnp.int32),
        pltpu.VMEM((CHUNK,), jnp.int32),
        pltpu.VMEM((CHUNK,), jnp.int32),
        pltpu.VMEM((CHUNK,), jnp.float32),
        pltpu.VMEM((CHUNK,), jnp.int32),
        pltpu.VMEM_SHARED((HST,), jnp.float32),
        pltpu.SemaphoreType.DMA,
    ],
  )(_bin_body)


def _bin_body(et_hbm, src_hbm, dst_hbm, rank_hbm, pp_hbm, ones_hbm, zeros_hbm,
              srcp_hbm, dstp_hbm, segp_hbm, cnt_hbm,
              et_c, src_c, dst_c, rank_c, pos_c, seg_c, cidx_c, ones_v,
              ppv_c, cnt_sh, semp):
    c = lax.axis_index("c")
    s = lax.axis_index("s")
    pltpu.sync_copy(zeros_hbm, cnt_sh.at[pl.ds(s * HZR, HZR)])
    pltpu.sync_copy(ones_hbm, ones_v)
    plsc.subcore_barrier()
    chs = c * HS

    def body(i, carry):
        base = (s * BPS + i) * CHUNK
        pltpu.sync_copy(et_hbm.at[pl.ds(base, CHUNK)], et_c)
        pltpu.sync_copy(dst_hbm.at[pl.ds(base, CHUNK)], dst_c)
        for j in range(CHUNK // 16):
            sl = pl.ds(j * 16, 16)
            etv = et_c[sl]
            segv = dst_c[sl] * R + etv
            seg_c[sl] = segv
            loc = segv - chs
            okc = jnp.logical_and(loc >= 0, loc < HS)
            okc = jnp.logical_and(okc, etv != 127)
            cidx_c[sl] = jnp.where(okc, loc, HS)
        pltpu.sync_copy(ones_v, cnt_sh.at[cidx_c], add=True)

        do_scat = jnp.logical_or(jnp.logical_and(c == 0, i < BPS2),
                                 jnp.logical_and(c == 1, i >= BPS2))

        @pl.when(do_scat)
        def _():
            pltpu.sync_copy(src_hbm.at[pl.ds(base, CHUNK)], src_c)
            pltpu.sync_copy(rank_hbm.at[pl.ds(base, CHUNK)], rank_c)
            pltpu.async_copy(pp_hbm.at[et_c], ppv_c, semp).wait()
            for j in range(CHUNK // 16):
                sl = pl.ds(j * 16, 16)
                pos_c[sl] = ppv_c[sl] + rank_c[sl]
            pltpu.sync_copy(src_c, srcp_hbm.at[pos_c])
            pltpu.sync_copy(dst_c, dstp_hbm.at[pos_c])
            pltpu.sync_copy(seg_c, segp_hbm.at[pos_c])
        return carry

    lax.fori_loop(0, BPS, body, 0)
    plsc.subcore_barrier()
    pltpu.sync_copy(cnt_sh.at[pl.ds(s * HCP, HCP)],
                    cnt_hbm.at[pl.ds(chs + s * HCP, HCP)])


@functools.lru_cache(maxsize=None)
def _norm2_kernel():
  return functools.partial(
    pl.kernel,
    out_type=jax.ShapeDtypeStruct((EPAD,), jnp.float32),
    mesh=_mesh(),
    scratch_types=[
        pltpu.VMEM((CHUNK,), jnp.int32),
        pltpu.VMEM((CHUNK,), jnp.float32),
        pltpu.VMEM((CHUNK,), jnp.float32),
        pltpu.SemaphoreType.DMA,
    ],
  )(_norm2_body)


def _norm2_body(seg_hbm, cnt_hbm, norm_hbm, idx_v, val_v, norm_v, sem):
    c = lax.axis_index("c")
    s = lax.axis_index("s")
    wid = s * 2 + c

    def body(i, carry):
        base = (wid * CPW + i) * CHUNK
        pltpu.sync_copy(seg_hbm.at[pl.ds(base, CHUNK)], idx_v)
        for j in range(CHUNK // 16):
            sl = pl.ds(j * 16, 16)
            idx_v[sl] = jnp.clip(idx_v[sl], 0, MSEG - 1)
        pltpu.async_copy(cnt_hbm.at[idx_v], val_v, sem).wait()
        for j in range(CHUNK // 16):
            sl = pl.ds(j * 16, 16)
            norm_v[sl] = 1.0 / jnp.maximum(val_v[sl], 1.0)
        pltpu.sync_copy(norm_v, norm_hbm.at[pl.ds(base, CHUNK)])
        return carry

    lax.fori_loop(0, CPW, body, 0)


@functools.lru_cache(maxsize=None)
def _gather_kernel():
  return functools.partial(
    pl.kernel,
    out_type=jax.ShapeDtypeStruct((EPAD, F), jnp.float32),
    mesh=_mesh(),
    scratch_types=[
        pltpu.VMEM((2, GRP, CHUNK), jnp.int32),
        pltpu.VMEM((2, GRP, CHUNK, F), jnp.float32),
        pltpu.SemaphoreType.DMA,
        pltpu.SemaphoreType.DMA,
        pltpu.SemaphoreType.DMA,
    ],
  )(_gather_body)


def _gather_body(tbl_hbm, src_hbm, out_hbm, idx_v, rows_v, semi, semg, semw):
    c = lax.axis_index("c")
    s = lax.axis_index("s")
    wid = s * 2 + c
    base0 = wid * CPW * CHUNK

    for b in range(GRP):
        pltpu.async_copy(src_hbm.at[pl.ds(base0 + b * CHUNK, CHUNK)],
                         idx_v.at[0, b], semi)

    def body(g, carry):
        par = g % 2
        gbase = base0 + g * GRP * CHUNK

        for b in range(GRP):
            pltpu.make_async_copy(
                src_hbm.at[pl.ds(base0, CHUNK)], idx_v.at[par, b], semi).wait()
        for b in range(GRP):
            for j in range(CHUNK // 16):
                sl = pl.ds(j * 16, 16)
                idx_v[par, b, sl] = jnp.clip(idx_v[par, b, sl], 0, N - 1)

        @pl.when(g >= 2)
        def _():
            for b in range(GRP):
                pltpu.make_async_copy(
                    rows_v.at[par, b], out_hbm.at[pl.ds(base0, CHUNK)],
                    semw).wait()
        for b in range(GRP):
            pltpu.async_copy(tbl_hbm.at[idx_v.at[par, b]],
                             rows_v.at[par, b], semg)

        @pl.when(g + 1 < NG)
        def _():
            nbase = gbase + GRP * CHUNK
            for b in range(GRP):
                pltpu.async_copy(src_hbm.at[pl.ds(nbase + b * CHUNK, CHUNK)],
                                 idx_v.at[1 - par, b], semi)

        for b in range(GRP):
            pltpu.make_async_copy(
                tbl_hbm.at[idx_v.at[par, b]], rows_v.at[par, b], semg).wait()
        for b in range(GRP):
            pltpu.async_copy(rows_v.at[par, b],
                             out_hbm.at[pl.ds(gbase + b * CHUNK, CHUNK)], semw)
        return carry

    lax.fori_loop(0, NG, body, 0)
    for b in range(GRP):
        pltpu.make_async_copy(
            rows_v.at[0, b], out_hbm.at[pl.ds(base0, CHUNK)], semw).wait()
        pltpu.make_async_copy(
            rows_v.at[1, b], out_hbm.at[pl.ds(base0, CHUNK)], semw).wait()


@functools.lru_cache(maxsize=None)
def _scatter_kernel():
  return functools.partial(
    pl.kernel,
    out_type=jax.ShapeDtypeStruct((NPAD, H), jnp.float32),
    mesh=_mesh(),
    scratch_types=[
        pltpu.VMEM((2, GRS, CHUNK), jnp.int32),
        pltpu.VMEM((2, GRS, CHUNK, H), jnp.float32),
        pltpu.VMEM_SHARED((ACCR, H), jnp.float32),
        pltpu.SemaphoreType.DMA,
        pltpu.SemaphoreType.DMA,
    ],
  )(_scatter_body)


def _scatter_body(msg_hbm, dst_hbm, zrows_hbm, out_hbm, idx_v, rows_v,
                  acc_sh, semi, semr):
    c = lax.axis_index("c")
    s = lax.axis_index("s")
    nbase_c = c * NHALF
    base0 = s * CPS2 * CHUNK
    pltpu.sync_copy(zrows_hbm, acc_sh.at[pl.ds(s * AZR, AZR)])
    plsc.subcore_barrier()

    for b in range(GRS):
        pltpu.async_copy(dst_hbm.at[pl.ds(base0 + b * CHUNK, CHUNK)],
                         idx_v.at[0, b], semi)
        pltpu.async_copy(msg_hbm.at[pl.ds(base0 + b * CHUNK, CHUNK)],
                         rows_v.at[0, b], semr)

    def body(g, carry):
        par = g % 2
        gbase = base0 + g * GRS * CHUNK
        for b in range(GRS):
            pltpu.make_async_copy(
                dst_hbm.at[pl.ds(base0, CHUNK)], idx_v.at[par, b], semi).wait()
            pltpu.make_async_copy(
                msg_hbm.at[pl.ds(base0, CHUNK)], rows_v.at[par, b],
                semr).wait()

        @pl.when(g + 1 < NGS2)
        def _():
            nbase = gbase + GRS * CHUNK
            for b in range(GRS):
                pltpu.async_copy(dst_hbm.at[pl.ds(nbase + b * CHUNK, CHUNK)],
                                 idx_v.at[1 - par, b], semi)
                pltpu.async_copy(msg_hbm.at[pl.ds(nbase + b * CHUNK, CHUNK)],
                                 rows_v.at[1 - par, b], semr)

        for b in range(GRS):
            for j in range(CHUNK // 16):
                sl = pl.ds(j * 16, 16)
                dv = idx_v[par, b, sl] - nbase_c
                ok = jnp.logical_and(dv >= 0, dv < NHALF)
                idx_v[par, b, sl] = jnp.where(ok, dv, NHALF)
            pltpu.sync_copy(rows_v.at[par, b], acc_sh.at[idx_v.at[par, b]],
                            add=True)
        return carry

    lax.fori_loop(0, NGS2, body, 0)
    plsc.subcore_barrier()
    pltpu.sync_copy(acc_sh.at[pl.ds(s * ACR, ACR)],
                    out_hbm.at[pl.ds(nbase_c + s * ACR, ACR)])


def _mm_body(rel_ref, end_ref, xg_ref, w_ref, nrm_ref, out_ref):
    t = pl.program_id(0)
    for sub in range(SUBT):
        r = rel_ref[t * SUBT + sub]
        rows = t * SUPB + sub * T + lax.broadcasted_iota(jnp.int32, (T, 1), 0)
        mask = (rows < end_ref[t * SUBT + sub]).astype(jnp.float32)
        out_ref[pl.ds(sub * T, T), :] = jnp.dot(
            xg_ref[pl.ds(sub * T, T), :], w_ref[r],
            preferred_element_type=jnp.float32) * (
                nrm_ref[pl.ds(sub * T, T), :] * mask)


def _mm(rel_of_tile, end_of_tile, xg, W, norm2):
    return pl.pallas_call(
        _mm_body,
        grid_spec=pltpu.PrefetchScalarGridSpec(
            num_scalar_prefetch=2,
            grid=(NSUP,),
            in_specs=[
                pl.BlockSpec((SUPB, F), lambda t, rel, end: (t, 0)),
                pl.BlockSpec((R, F, H), lambda t, rel, end: (0, 0, 0)),
                pl.BlockSpec((SUPB, 1), lambda t, rel, end: (t, 0)),
            ],
            out_specs=pl.BlockSpec((SUPB, H), lambda t, rel, end: (t, 0)),
        ),
        out_shape=jax.ShapeDtypeStruct((EPAD, H), jnp.float32),
    )(rel_of_tile, end_of_tile, xg, W, norm2)


def _update_body(p0_ref, h_ref, root_ref, b_ref, out_ref):
    acc = p0_ref[...] + jnp.dot(
        h_ref[...], root_ref[...], preferred_element_type=jnp.float32)
    out_ref[...] = jnp.maximum(acc + b_ref[...], 0.0)


def _update(p0, h, root, b2d):
    return pl.pallas_call(
        _update_body,
        grid=(N // _UROWS,),
        in_specs=[
            pl.BlockSpec((_UROWS, H), lambda t: (t, 0)),
            pl.BlockSpec((_UROWS, F), lambda t: (t, 0)),
            pl.BlockSpec((F, H), lambda t: (0, 0)),
            pl.BlockSpec((1, H), lambda t: (0, 0)),
        ],
        out_specs=pl.BlockSpec((_UROWS, H), lambda t: (t, 0)),
        out_shape=jax.ShapeDtypeStruct((N, H), jnp.float32),
    )(p0, h, root, b2d)


def _pool_body(h_ref, batch_ref, wsw_ref, wsb_ref, w1_ref, b1_ref,
               w2_ref, b2_ref, w3_ref, b3_ref, ow_ref, ob_ref,
               out_ref, acc_ref):
    t = pl.program_id(0)

    @pl.when(t == 0)
    def _():
        acc_ref[...] = jnp.zeros_like(acc_ref)

    z = jnp.dot(h_ref[...], wsw_ref[...],
                preferred_element_type=jnp.float32) + wsb_ref[0, 0]
    w = 1.0 / (1.0 + jnp.exp(-z))
    wh = h_ref[...] * w
    onehot = (batch_ref[...] == lax.broadcasted_iota(
        jnp.int32, (1, G), 1)).astype(jnp.float32)
    acc_ref[...] += lax.dot_general(
        onehot, wh, (((0,), (0,)), ((), ())),
        preferred_element_type=jnp.float32,
        precision=lax.Precision.HIGHEST)

    @pl.when(t == N // _UROWS - 1)
    def _():
        g = acc_ref[...]
        m = jnp.maximum(jnp.dot(g, w1_ref[...],
                                preferred_element_type=jnp.float32)
                        + b1_ref[...], 0.0)
        m = jnp.maximum(jnp.dot(m, w2_ref[...],
                                preferred_element_type=jnp.float32)
                        + b2_ref[...], 0.0)
        m = jnp.dot(m, w3_ref[...],
                    preferred_element_type=jnp.float32) + b3_ref[...]
        out_ref[...] = jnp.dot(m, ow_ref[...],
                               preferred_element_type=jnp.float32) + ob_ref[0, 0]


def _pool(h, batch2, ws_w, wsb2, m_w1, mb1, m_w2, mb2, m_w3, mb3, out_w, ob2):
    return pl.pallas_call(
        _pool_body,
        grid=(N // _UROWS,),
        in_specs=[
            pl.BlockSpec((_UROWS, H), lambda t: (t, 0)),
            pl.BlockSpec((_UROWS, 1), lambda t: (t, 0)),
            pl.BlockSpec((H, 1), lambda t: (0, 0)),
            pl.BlockSpec((1, 1), lambda t: (0, 0)),
            pl.BlockSpec((H, MLP_H), lambda t: (0, 0)),
            pl.BlockSpec((1, MLP_H), lambda t: (0, 0)),
            pl.BlockSpec((MLP_H, MLP_H), lambda t: (0, 0)),
            pl.BlockSpec((1, MLP_H), lambda t: (0, 0)),
            pl.BlockSpec((MLP_H, MLP_H), lambda t: (0, 0)),
            pl.BlockSpec((1, MLP_H), lambda t: (0, 0)),
            pl.BlockSpec((MLP_H, 1), lambda t: (0, 0)),
            pl.BlockSpec((1, 1), lambda t: (0, 0)),
        ],
        out_specs=pl.BlockSpec((G, 1), lambda t: (0, 0)),
        out_shape=jax.ShapeDtypeStruct((G, 1), jnp.float32),
        scratch_shapes=[pltpu.VMEM((G, H), jnp.float32)],
    )(h, batch2, ws_w, wsb2, m_w1, mb1, m_w2, mb2, m_w3, mb3, out_w, ob2)


@jax.jit
def _run(x, edge_index, edge_type, batch, W1, root1, b1, W2, root2, b2,
         ws_w, ws_b, m_w1, m_b1, m_w2, m_b2, m_w3, m_b3, out_w, out_b):
    loops = jnp.arange(N, dtype=jnp.int32)
    zpad = jnp.zeros(EIN - ETOT, jnp.int32)
    src = jnp.concatenate([edge_index[0].astype(jnp.int32), loops, zpad])
    dst = jnp.concatenate([edge_index[1].astype(jnp.int32), loops, zpad])
    et = jnp.concatenate([edge_type.reshape(-1).astype(jnp.int32),
                          jnp.zeros(N, jnp.int32),
                          jnp.full(EIN - ETOT, 127, jnp.int32)])

    rank2, cnt128 = _rank(et.reshape(EIN, 1))
    rank = rank2.reshape(EIN)
    cnts = cnt128[0, :R].astype(jnp.int32)
    cp = ((cnts + T - 1) // T) * T
    ppr = jnp.concatenate(
        [jnp.zeros(1, jnp.int32), jnp.cumsum(cp)[:-1].astype(jnp.int32)])
    pp_tbl = jnp.concatenate(
        [ppr, jnp.full(128 - R, EPAD, jnp.int32)])
    tstart = jnp.arange(NT, dtype=jnp.int32) * T
    rel_of_tile = jnp.clip(
        (tstart[:, None] >= ppr[None, :]).sum(1).astype(jnp.int32) - 1,
        0, R - 1)
    ends = ppr + cnts
    oh_rel = (rel_of_tile[:, None] ==
              jnp.arange(R, dtype=jnp.int32)[None, :])
    end_of_tile = jnp.where(oh_rel, ends[None, :], 0).sum(1).astype(jnp.int32)

    ones_c = jnp.ones((CHUNK,), jnp.float32)
    zeros_h = jnp.zeros((HZR,), jnp.float32)
    zeros_r = jnp.zeros((AZR, H), jnp.float32)

    src_px, dst_px, seg_px, cnt = _bin_kernel()(
        et, src, dst, rank, pp_tbl, ones_c, zeros_h)
    src_p = src_px[:EPAD]
    dst_p = dst_px[:EPAD]
    seg_p = seg_px[:EPAD]
    norm = _norm2_kernel()(seg_p, cnt)
    norm2 = norm.reshape(EPAD, 1)

    h = x
    for (Wl, rootl, bl) in ((W1, root1, b1), (W2, root2, b2)):
        xg = _gather_kernel()(h, src_p)
        msg = _mm(rel_of_tile, end_of_tile, xg, Wl, norm2)
        parts = _scatter_kernel()(msg, dst_p, zeros_r)
        h = _update(parts[:N], h, rootl, bl.reshape(1, H))
    return _pool(h, batch.reshape(N, 1), ws_w, ws_b.reshape(1, 1),
                 m_w1, m_b1.reshape(1, MLP_H), m_w2, m_b2.reshape(1, MLP_H),
                 m_w3, m_b3.reshape(1, MLP_H), out_w, out_b.reshape(1, 1))


def kernel(x, edge_index, edge_type, batch, W1, root1, b1, W2, root2, b2,
           ws_w, ws_b, m_w1, m_b1, m_w2, m_b2, m_w3, m_b3, out_w, out_b):
    return _run(x, edge_index, edge_type, batch, W1, root1, b1, W2, root2, b2,
                ws_w, ws_b, m_w1, m_b1, m_w2, m_b2, m_w3, m_b3, out_w, out_b)
